# Initial kernel scaffold; baseline (speedup 1.0000x reference)
import jax
import jax.numpy as jnp
from jax.experimental import pallas as pl


def kernel(x, table, W, b):
    raise NotImplementedError("write your pallas kernel here")


# speedup vs baseline: 95.9646x; 95.9646x over previous
import functools

import jax
import jax.numpy as jnp
from jax import lax
from jax.experimental import pallas as pl
from jax.experimental.pallas import tpu as pltpu
from jax.experimental.pallas import tpu_sc as plsc

_B, _L = 16384, 200
_N = _B * _L
_BLK = 12800
_UNROLL = 4


def _score_gather(xf, tcol, wrow, bvec):
    mesh = plsc.VectorSubcoreMesh(core_axis_name="c", subcore_axis_name="s")

    @functools.partial(
        pl.kernel,
        out_type=jax.ShapeDtypeStruct((_N,), jnp.float32),
        mesh=mesh,
        scratch_types=[
            pltpu.VMEM((8, 16), jnp.float32),
            pltpu.VMEM((8, 16), jnp.float32),
            pltpu.VMEM((16,), jnp.float32),
            pltpu.SemaphoreType.DMA,
        ],
    )
    def run(x_hbm, t_hbm, w_hbm, b_hbm, o_hbm, t_v, w_v, b_v, sem):
        pltpu.async_copy(t_hbm, t_v, sem).wait()
        pltpu.async_copy(w_hbm, w_v, sem).wait()
        pltpu.async_copy(b_hbm, b_v, sem).wait()

        scores = b_v[...]
        for d in range(5):
            scores = scores + t_v[d] * w_v[d]

        dnums = lax.GatherDimensionNumbers(
            offset_dims=(), collapsed_slice_dims=(0,), start_index_map=(0,)
        )

        def body(x_vmem, o_vmem):
            @pl.loop(0, _BLK, step=16 * _UNROLL)
            def _(j):
                for u in range(_UNROLL):
                    idx = x_vmem[pl.ds(j + 16 * u, 16)]
                    o_vmem[pl.ds(j + 16 * u, 16)] = lax.gather(
                        scores,
                        idx[:, None],
                        dnums,
                        slice_sizes=(1,),
                        mode=lax.GatherScatterMode.PROMISE_IN_BOUNDS,
                    )

        pltpu.emit_pipeline(
            body,
            grid=(_N // _BLK,),
            in_specs=[pl.BlockSpec((_BLK,), lambda i: (i,))],
            out_specs=[pl.BlockSpec((_BLK,), lambda i: (i,))],
            core_axis_name=("c", "s"),
            dimension_semantics=(pltpu.PARALLEL,),
        )(x_hbm, o_hbm)

    return run(xf, tcol, wrow, bvec)


def kernel(x, table, W, b):
    xf = x.reshape(_N)
    tcol = jnp.zeros((8, 16), jnp.float32).at[:5, :10].set(table.T)
    wrow = jnp.zeros((8, 16), jnp.float32).at[:5, :].set(
        jnp.broadcast_to(W.reshape(5, 1), (5, 16))
    )
    bvec = jnp.broadcast_to(b, (16,)).astype(jnp.float32)
    out = _score_gather(xf, tcol, wrow, bvec)
    return out.reshape(_B, _L, 1)


# speedup vs baseline: 118.6649x; 1.2365x over previous
import functools

import jax
import jax.numpy as jnp
from jax import lax
from jax.experimental import pallas as pl
from jax.experimental.pallas import tpu as pltpu
from jax.experimental.pallas import tpu_sc as plsc

_B, _L = 16384, 200
_N = _B * _L
_BLK = 12800
_UNROLL = 8


def _score_gather(xf, tcol, wrow, bvec):
    mesh = plsc.VectorSubcoreMesh(core_axis_name="c", subcore_axis_name="s")

    @functools.partial(
        pl.kernel,
        out_type=jax.ShapeDtypeStruct((_N,), jnp.float32),
        mesh=mesh,
        scratch_types=[
            pltpu.VMEM((8, 16), jnp.float32),
            pltpu.VMEM((8, 16), jnp.float32),
            pltpu.VMEM((16,), jnp.float32),
            pltpu.SemaphoreType.DMA,
        ],
    )
    def run(x_hbm, t_hbm, w_hbm, b_hbm, o_hbm, t_v, w_v, b_v, sem):
        pltpu.async_copy(t_hbm, t_v, sem).wait()
        pltpu.async_copy(w_hbm, w_v, sem).wait()
        pltpu.async_copy(b_hbm, b_v, sem).wait()

        scores = b_v[...]
        for d in range(5):
            scores = scores + t_v[d] * w_v[d]

        dnums = lax.GatherDimensionNumbers(
            offset_dims=(), collapsed_slice_dims=(0,), start_index_map=(0,)
        )

        def body(x_vmem, o_vmem):
            @plsc.parallel_loop(0, _BLK, step=16, unroll=_UNROLL)
            def _(j):
                idx = x_vmem[pl.ds(j, 16)]
                o_vmem[pl.ds(j, 16)] = lax.gather(
                    scores,
                    idx[:, None],
                    dnums,
                    slice_sizes=(1,),
                    mode=lax.GatherScatterMode.PROMISE_IN_BOUNDS,
                )

        pltpu.emit_pipeline(
            body,
            grid=(_N // _BLK,),
            in_specs=[pl.BlockSpec((_BLK,), lambda i: (i,))],
            out_specs=[pl.BlockSpec((_BLK,), lambda i: (i,))],
            core_axis_name=("c", "s"),
            dimension_semantics=(pltpu.PARALLEL,),
        )(x_hbm, o_hbm)

    return run(xf, tcol, wrow, bvec)


def kernel(x, table, W, b):
    xf = x.reshape(_N)
    tcol = jnp.zeros((8, 16), jnp.float32).at[:5, :10].set(table.T)
    wrow = jnp.zeros((8, 16), jnp.float32).at[:5, :].set(
        jnp.broadcast_to(W.reshape(5, 1), (5, 16))
    )
    bvec = jnp.broadcast_to(b, (16,)).astype(jnp.float32)
    out = _score_gather(xf, tcol, wrow, bvec)
    return out.reshape(_B, _L, 1)


# speedup vs baseline: 202.7293x; 1.7084x over previous
import functools

import jax
import jax.numpy as jnp
from jax import lax
from jax.experimental import pallas as pl
from jax.experimental.pallas import tpu as pltpu
from jax.experimental.pallas import tpu_sc as plsc

_B, _L = 16384, 200
_BR = 64
_UNROLL = 2
_COLS = list(range(0, 192, 16)) + [184]


def _score_gather(x, tw, ww):
    mesh = plsc.VectorSubcoreMesh(core_axis_name="c", subcore_axis_name="s")

    @functools.partial(
        pl.kernel,
        out_type=jax.ShapeDtypeStruct((_B, _L), jnp.float32),
        mesh=mesh,
        compiler_params=pltpu.CompilerParams(use_tc_tiling_on_sc=True),
        scratch_types=[
            pltpu.VMEM((8, 128), jnp.float32),
            pltpu.VMEM((8, 128), jnp.float32),
            pltpu.SemaphoreType.DMA,
        ],
    )
    def run(x_hbm, t_hbm, w_hbm, o_hbm, t_v, w_v, sem):
        pltpu.async_copy(t_hbm, t_v, sem).wait()
        pltpu.async_copy(w_hbm, w_v, sem).wait()

        scores = t_v[0, pl.ds(0, 16)] * w_v[0, pl.ds(0, 16)]
        for d in range(1, 6):
            scores = scores + t_v[d, pl.ds(0, 16)] * w_v[d, pl.ds(0, 16)]

        dnums = lax.GatherDimensionNumbers(
            offset_dims=(), collapsed_slice_dims=(0,), start_index_map=(0,)
        )

        def body(x_vmem, o_vmem):
            @plsc.parallel_loop(0, _BR, step=1, unroll=_UNROLL)
            def _(r):
                for c in _COLS:
                    idx = x_vmem[r, pl.ds(c, 16)]
                    o_vmem[r, pl.ds(c, 16)] = lax.gather(
                        scores,
                        idx[:, None],
                        dnums,
                        slice_sizes=(1,),
                        mode=lax.GatherScatterMode.PROMISE_IN_BOUNDS,
                    )

        pltpu.emit_pipeline(
            body,
            grid=(_B // _BR,),
            in_specs=[pl.BlockSpec((_BR, _L), lambda i: (i, 0))],
            out_specs=[pl.BlockSpec((_BR, _L), lambda i: (i, 0))],
            core_axis_name=("c", "s"),
            dimension_semantics=(pltpu.PARALLEL,),
        )(x_hbm, o_hbm)

    return run(x, tw, ww)


def kernel(x, table, W, b):
    tw = (
        jnp.zeros((8, 128), jnp.float32)
        .at[:5, :10]
        .set(table.T)
        .at[5, :]
        .set(1.0)
    )
    ww = (
        jnp.zeros((8, 128), jnp.float32)
        .at[:5, :]
        .set(jnp.broadcast_to(W.reshape(5, 1), (5, 128)))
        .at[5, :]
        .set(b[0])
    )
    out = _score_gather(x, tw, ww)
    return out.reshape(_B, _L, 1)
